# baseline (speedup 1.0000x reference)
import functools

import jax
import jax.numpy as jnp
from jax import lax
from jax.experimental import pallas as pl
from jax.experimental.pallas import tpu as pltpu
from jax.experimental.pallas import tpu_sc as plsc

N = 10000
E = 320000
D = 128
R = 9
H = 128

NC = 2
NS = 16
NW = NC * NS

CH = 10
EC = E // CH

_SC_MESH = plsc.VectorSubcoreMesh(core_axis_name="c", subcore_axis_name="s")

GRPW = 2 * EC // NW
GCH = 128
GFULL = GRPW // GCH
GTAIL = GRPW - GFULL * GCH
GPAIRS = GFULL // 2


@functools.partial(
    pl.kernel,
    out_type=jax.ShapeDtypeStruct((2 * EC, D), jnp.float32),
    mesh=_SC_MESH,
    scratch_types=[
        pltpu.VMEM((4, GCH), jnp.int32),
        pltpu.VMEM((4, GCH, D), jnp.float32),
        pltpu.VMEM((GTAIL,), jnp.int32),
        pltpu.VMEM((GTAIL, D), jnp.float32),
        pltpu.SemaphoreType.DMA,
        pltpu.SemaphoreType.DMA,
        pltpu.SemaphoreType.DMA,
        pltpu.SemaphoreType.DMA,
    ],
)
def _sc_gather(idx_hbm, node_hbm, out_hbm, idx_v, rows_v, tidx_v, trows_v,
               gA, gB, wA, wB):
    cid = lax.axis_index("c")
    sid = lax.axis_index("s")
    base = (sid * NC + cid) * GRPW

    def wait_gather(slot, sem):
        pltpu.make_async_copy(node_hbm.at[idx_v.at[slot]], rows_v.at[slot], sem).wait()

    def drain_write(sem):
        pltpu.make_async_copy(rows_v.at[0], out_hbm.at[pl.ds(base, GCH)], sem).wait()

    def pair(m, carry):
        sA = 2 * lax.rem(m, 2)
        pA = 2 * lax.rem(m + 1, 2)
        offA = base + 2 * m * GCH

        @pl.when(m >= 1)
        def _retire_a():
            wait_gather(pA, gA)
            pltpu.async_copy(rows_v.at[pA],
                             out_hbm.at[pl.ds(offA - 2 * GCH, GCH)], wA)

        @pl.when(m >= 2)
        def _drain_a():
            drain_write(wA)

        pltpu.sync_copy(idx_hbm.at[pl.ds(offA, GCH)], idx_v.at[sA])
        pltpu.async_copy(node_hbm.at[idx_v.at[sA]], rows_v.at[sA], gA)

        @pl.when(m >= 1)
        def _retire_b():
            wait_gather(pA + 1, gB)
            pltpu.async_copy(rows_v.at[pA + 1],
                             out_hbm.at[pl.ds(offA - GCH, GCH)], wB)

        @pl.when(m >= 2)
        def _drain_b():
            drain_write(wB)

        pltpu.sync_copy(idx_hbm.at[pl.ds(offA + GCH, GCH)], idx_v.at[sA + 1])
        pltpu.async_copy(node_hbm.at[idx_v.at[sA + 1]], rows_v.at[sA + 1], gB)
        return carry

    lax.fori_loop(0, GPAIRS, pair, 0)
    lastA = 2 * GPAIRS - 2
    slA = lax.rem(lastA, 4)
    wait_gather(slA, gA)
    pltpu.async_copy(rows_v.at[slA], out_hbm.at[pl.ds(base + lastA * GCH, GCH)], wA)
    wait_gather(slA + 1, gB)
    pltpu.async_copy(rows_v.at[slA + 1],
                     out_hbm.at[pl.ds(base + (lastA + 1) * GCH, GCH)], wB)
    drain_write(wA)
    kL = 2 * GPAIRS
    sL = lax.rem(kL, 4)
    offL = base + kL * GCH
    pltpu.sync_copy(idx_hbm.at[pl.ds(offL, GCH)], idx_v.at[sL])
    pltpu.async_copy(node_hbm.at[idx_v.at[sL]], rows_v.at[sL], gA)
    wait_gather(sL, gA)
    pltpu.async_copy(rows_v.at[sL], out_hbm.at[pl.ds(offL, GCH)], wA)
    toff = base + GFULL * GCH
    pltpu.sync_copy(idx_hbm.at[pl.ds(toff, GTAIL)], tidx_v)
    pltpu.async_copy(node_hbm.at[tidx_v], trows_v, gA).wait()
    pltpu.sync_copy(trows_v, out_hbm.at[pl.ds(toff, GTAIL)])
    drain_write(wA)
    drain_write(wA)
    drain_write(wB)
    drain_write(wB)


BE = 3200
CBLK = EC // BE


def _sigmoid(x):
    return 0.5 + 0.5 * jnp.tanh(0.5 * x)


def _silu(x):
    u = 0.5 * x
    return u * (1.0 + jnp.tanh(u))


def _mlp_body(vi_ref, vj_ref, ef_ref, rbft_ref,
              W1vi, W1vj, W1ef, b1, Wfix,
              eW2, eb2, eG2, eg2, nW2, nb2, nG2, ng2,
              We, Wv, enew_ref, mess_ref):
    bf = jnp.bfloat16
    f32 = jnp.float32
    ef = ef_ref[...]
    vi_bf = vi_ref[...].astype(bf)
    vj_bf = vj_ref[...].astype(bf)
    ef_bf = ef.astype(bf)
    rbft = rbft_ref[...]
    dn_t = (((0,), (0,)), ((), ()))

    Z = (jnp.dot(vi_bf, W1vi[...], preferred_element_type=f32)
         + jnp.dot(vj_bf, W1vj[...], preferred_element_type=f32)
         + jnp.dot(ef_bf, W1ef[...], preferred_element_type=f32)
         + b1[...])

    h = _silu(Z[:, 0 * H:1 * H])
    h = _silu(jnp.dot(h.astype(bf), eW2[...], preferred_element_type=f32) + eb2[...])
    g = _silu(Z[:, 1 * H:2 * H])
    g = _sigmoid(jnp.dot(g.astype(bf), eG2[...], preferred_element_type=f32) + eg2[...])
    rwe = lax.dot_general(rbft, We[...], dn_t, preferred_element_type=f32)
    mij = h * g * rwe
    enew_ref[...] = ef + mij

    F = jnp.dot(mij.astype(bf), Wfix[...], preferred_element_type=f32)
    p = _silu(Z[:, 2 * H:3 * H] + F[:, 0 * H:1 * H])
    p = _silu(jnp.dot(p.astype(bf), nW2[...], preferred_element_type=f32) + nb2[...])
    q = _silu(Z[:, 3 * H:4 * H] + F[:, 1 * H:2 * H])
    q = _sigmoid(jnp.dot(q.astype(bf), nG2[...], preferred_element_type=f32) + ng2[...])
    rwv = lax.dot_general(rbft, Wv[...], dn_t, preferred_element_type=f32)
    mess_ref[...] = p * q * rwv


def _mlp_body_alias(vi_ref, vj_ref, ef_ref, rbft_ref,
                    W1vi, W1vj, W1ef, b1, Wfix,
                    eW2, eb2, eG2, eg2, nW2, nb2, nG2, ng2,
                    We, Wv, eprev_ref, enew_ref, mess_ref):
    del eprev_ref
    _mlp_body(vi_ref, vj_ref, ef_ref, rbft_ref,
              W1vi, W1vj, W1ef, b1, Wfix,
              eW2, eb2, eG2, eg2, nW2, nb2, nG2, ng2,
              We, Wv, enew_ref, mess_ref)


def _fixed(i):
    return (0, 0)


def _tc_mlp_chunk(c, vivj, edge_feat, rbft, weights, e_prev):
    off = c * CBLK
    in_specs = [
        pl.BlockSpec((BE, D), lambda i: (i, 0)),
        pl.BlockSpec((BE, D), lambda i: (CBLK + i, 0)),
        pl.BlockSpec((BE, D), lambda i, off=off: (off + i, 0)),
        pl.BlockSpec((R, BE), lambda i, off=off: (0, off + i)),
        *[pl.BlockSpec(w.shape, _fixed) for w in weights],
    ]
    out_specs = [
        pl.BlockSpec((BE, D), lambda i, off=off: (off + i, 0)),
        pl.BlockSpec((BE, D), lambda i: (i, 0)),
    ]
    out_shape = [
        jax.ShapeDtypeStruct((E, D), jnp.float32),
        jax.ShapeDtypeStruct((EC, D), jnp.float32),
    ]
    args = [vivj, vivj, edge_feat, rbft, *weights]
    if e_prev is None:
        body = _mlp_body
        aliases = {}
    else:
        body = _mlp_body_alias
        in_specs.append(pl.BlockSpec(memory_space=pl.ANY))
        args.append(e_prev)
        aliases = {len(args) - 1: 0}
    return pl.pallas_call(
        body,
        grid=(CBLK,),
        in_specs=in_specs,
        out_specs=out_specs,
        out_shape=out_shape,
        input_output_aliases=aliases,
        compiler_params=pltpu.CompilerParams(
            dimension_semantics=("arbitrary",),
        ),
    )(*args)


SCH = 128
NSTEP = EC // SCH
SBASE = NSTEP // NW
SEXTRA = NSTEP - SBASE * NW
NPS = 632
N_PAD = NPS * NS


@functools.partial(
    pl.kernel,
    out_type=jax.ShapeDtypeStruct((2 * N_PAD, D), jnp.float32),
    mesh=_SC_MESH,
    scratch_types=[
        pltpu.VMEM((2, SCH), jnp.int32),
        pltpu.VMEM((2, SCH, D), jnp.float32),
        pltpu.VMEM_SHARED((N_PAD, D), jnp.float32),
        pltpu.SemaphoreType.DMA,
        pltpu.SemaphoreType.DMA,
    ],
)
def _sc_scatter(mess_hbm, dst_hbm, init_hbm, out_hbm, idx_v, rows_v,
                acc_sh, lsem, ssem):
    cid = lax.axis_index("c")
    sid = lax.axis_index("s")
    pltpu.sync_copy(init_hbm.at[pl.ds(cid * N_PAD + sid * NPS, NPS)],
                    acc_sh.at[pl.ds(sid * NPS, NPS)])
    plsc.subcore_barrier()

    wid = sid * NC + cid
    nsteps = jnp.where(wid < SEXTRA, SBASE + 1, SBASE)

    def step(k, carry):
        b = lax.rem(k, 2)
        pb = 1 - b
        off = (wid + k * NW) * SCH

        @pl.when(k >= 2)
        def _drain_scatter():
            pltpu.make_async_copy(
                rows_v.at[b], acc_sh.at[idx_v.at[b]], ssem).wait()

        @pl.when(k >= 1)
        def _retire_prev():
            pltpu.make_async_copy(
                mess_hbm.at[pl.ds(0, SCH)], rows_v.at[pb], lsem).wait()
            pltpu.async_copy(rows_v.at[pb], acc_sh.at[idx_v.at[pb]], ssem,
                             add=True)

        pltpu.sync_copy(dst_hbm.at[pl.ds(off, SCH)], idx_v.at[b])
        pltpu.async_copy(mess_hbm.at[pl.ds(off, SCH)], rows_v.at[b], lsem)
        return carry

    lax.fori_loop(0, nsteps, step, 0)
    lb = lax.rem(nsteps - 1, 2)
    pltpu.make_async_copy(mess_hbm.at[pl.ds(0, SCH)], rows_v.at[lb], lsem).wait()
    pltpu.async_copy(rows_v.at[lb], acc_sh.at[idx_v.at[lb]], ssem, add=True)
    pltpu.make_async_copy(rows_v.at[0], acc_sh.at[idx_v.at[0]], ssem).wait()
    pltpu.make_async_copy(rows_v.at[0], acc_sh.at[idx_v.at[0]], ssem).wait()

    plsc.subcore_barrier()
    pltpu.sync_copy(acc_sh.at[pl.ds(sid * NPS, NPS)],
                    out_hbm.at[pl.ds(cid * N_PAD + sid * NPS, NPS)])


def kernel(node_feat, edge_feat, rbf, edge_index,
           eW1, eb1, eW2, eb2, eG1, eg1, eG2, eg2,
           nW1, nb1, nW2, nb2, nG1, ng1, nG2, ng2,
           We, Wv):
    src = edge_index[0].astype(jnp.int32)
    dst = edge_index[1].astype(jnp.int32)
    bf = jnp.bfloat16

    w1 = jnp.concatenate([eW1, eG1, nW1, nG1], axis=1).astype(bf)
    b1 = jnp.concatenate([eb1, eg1, nb1, ng1]).reshape(1, 4 * H)
    wfix = jnp.concatenate([nW1[2 * D:], nG1[2 * D:]], axis=1).astype(bf)
    weights = (w1[:D], w1[D:2 * D], w1[2 * D:], b1, wfix,
               eW2.astype(bf), eb2.reshape(1, H), eG2.astype(bf), eg2.reshape(1, H),
               nW2.astype(bf), nb2.reshape(1, H), nG2.astype(bf), ng2.reshape(1, H),
               We, Wv)
    rbft = rbf.T

    vivjs = [
        _sc_gather(jnp.concatenate([src[c * EC:(c + 1) * EC],
                                    dst[c * EC:(c + 1) * EC]]), node_feat)
        for c in range(CH)
    ]

    part = jnp.zeros((2 * N_PAD, D), jnp.float32).at[:N].set(node_feat)
    e_new = None
    for c in range(CH):
        e_new, mess = _tc_mlp_chunk(c, vivjs[c], edge_feat, rbft, weights, e_new)
        part = _sc_scatter(mess, dst[c * EC:(c + 1) * EC], part)

    v_new = part[:N] + part[N_PAD:N_PAD + N]
    return (e_new, v_new)


# speedup vs baseline: 1.0772x; 1.0772x over previous
import functools

import jax
import jax.numpy as jnp
from jax import lax
from jax.experimental import pallas as pl
from jax.experimental.pallas import tpu as pltpu
from jax.experimental.pallas import tpu_sc as plsc

N = 10000
E = 320000
D = 128
R = 9
H = 128

NC = 2
NS = 16
NW = NC * NS

CH = 5
EC = E // CH

_SC_MESH = plsc.VectorSubcoreMesh(core_axis_name="c", subcore_axis_name="s")

GRPW = 2 * EC // NW
GCH = 128
GFULL = GRPW // GCH
GTAIL = GRPW - GFULL * GCH
GPAIRS = GFULL // 2


@functools.partial(
    pl.kernel,
    out_type=jax.ShapeDtypeStruct((2 * EC, D), jnp.float32),
    mesh=_SC_MESH,
    scratch_types=[
        pltpu.VMEM((4, GCH), jnp.int32),
        pltpu.VMEM((4, GCH, D), jnp.float32),
        pltpu.VMEM((GTAIL,), jnp.int32),
        pltpu.VMEM((GTAIL, D), jnp.float32),
        pltpu.SemaphoreType.DMA,
        pltpu.SemaphoreType.DMA,
        pltpu.SemaphoreType.DMA,
        pltpu.SemaphoreType.DMA,
    ],
)
def _sc_gather(idx_hbm, node_hbm, out_hbm, idx_v, rows_v, tidx_v, trows_v,
               gA, gB, wA, wB):
    cid = lax.axis_index("c")
    sid = lax.axis_index("s")
    base = (sid * NC + cid) * GRPW

    def wait_gather(slot, sem):
        pltpu.make_async_copy(node_hbm.at[idx_v.at[slot]], rows_v.at[slot], sem).wait()

    def drain_write(sem):
        pltpu.make_async_copy(rows_v.at[0], out_hbm.at[pl.ds(base, GCH)], sem).wait()

    def pair(m, carry):
        sA = 2 * lax.rem(m, 2)
        pA = 2 * lax.rem(m + 1, 2)
        offA = base + 2 * m * GCH

        @pl.when(m >= 1)
        def _retire_a():
            wait_gather(pA, gA)
            pltpu.async_copy(rows_v.at[pA],
                             out_hbm.at[pl.ds(offA - 2 * GCH, GCH)], wA)

        @pl.when(m >= 2)
        def _drain_a():
            drain_write(wA)

        pltpu.sync_copy(idx_hbm.at[pl.ds(offA, GCH)], idx_v.at[sA])
        pltpu.async_copy(node_hbm.at[idx_v.at[sA]], rows_v.at[sA], gA)

        @pl.when(m >= 1)
        def _retire_b():
            wait_gather(pA + 1, gB)
            pltpu.async_copy(rows_v.at[pA + 1],
                             out_hbm.at[pl.ds(offA - GCH, GCH)], wB)

        @pl.when(m >= 2)
        def _drain_b():
            drain_write(wB)

        pltpu.sync_copy(idx_hbm.at[pl.ds(offA + GCH, GCH)], idx_v.at[sA + 1])
        pltpu.async_copy(node_hbm.at[idx_v.at[sA + 1]], rows_v.at[sA + 1], gB)
        return carry

    lax.fori_loop(0, GPAIRS, pair, 0)
    lastA = 2 * GPAIRS - 2
    slA = lax.rem(lastA, 4)
    wait_gather(slA, gA)
    pltpu.async_copy(rows_v.at[slA], out_hbm.at[pl.ds(base + lastA * GCH, GCH)], wA)
    wait_gather(slA + 1, gB)
    pltpu.async_copy(rows_v.at[slA + 1],
                     out_hbm.at[pl.ds(base + (lastA + 1) * GCH, GCH)], wB)
    drain_write(wA)
    kL = 2 * GPAIRS
    sL = lax.rem(kL, 4)
    offL = base + kL * GCH
    pltpu.sync_copy(idx_hbm.at[pl.ds(offL, GCH)], idx_v.at[sL])
    pltpu.async_copy(node_hbm.at[idx_v.at[sL]], rows_v.at[sL], gA)
    wait_gather(sL, gA)
    pltpu.async_copy(rows_v.at[sL], out_hbm.at[pl.ds(offL, GCH)], wA)
    toff = base + GFULL * GCH
    pltpu.sync_copy(idx_hbm.at[pl.ds(toff, GTAIL)], tidx_v)
    pltpu.async_copy(node_hbm.at[tidx_v], trows_v, gA).wait()
    pltpu.sync_copy(trows_v, out_hbm.at[pl.ds(toff, GTAIL)])
    drain_write(wA)
    drain_write(wA)
    drain_write(wB)
    drain_write(wB)


BE = 3200
CBLK = EC // BE


def _sigmoid(x):
    return 0.5 + 0.5 * jnp.tanh(0.5 * x)


def _silu(x):
    u = 0.5 * x
    return u * (1.0 + jnp.tanh(u))


def _mlp_body(vi_ref, vj_ref, ef_ref, rbft_ref,
              W1vi, W1vj, W1ef, b1, Wfix,
              eW2, eb2, eG2, eg2, nW2, nb2, nG2, ng2,
              We, Wv, enew_ref, mess_ref):
    bf = jnp.bfloat16
    f32 = jnp.float32
    ef = ef_ref[...]
    vi_bf = vi_ref[...].astype(bf)
    vj_bf = vj_ref[...].astype(bf)
    ef_bf = ef.astype(bf)
    rbft = rbft_ref[...]
    dn_t = (((0,), (0,)), ((), ()))

    Z = (jnp.dot(vi_bf, W1vi[...], preferred_element_type=f32)
         + jnp.dot(vj_bf, W1vj[...], preferred_element_type=f32)
         + jnp.dot(ef_bf, W1ef[...], preferred_element_type=f32)
         + b1[...])

    h = _silu(Z[:, 0 * H:1 * H])
    h = _silu(jnp.dot(h.astype(bf), eW2[...], preferred_element_type=f32) + eb2[...])
    g = _silu(Z[:, 1 * H:2 * H])
    g = _sigmoid(jnp.dot(g.astype(bf), eG2[...], preferred_element_type=f32) + eg2[...])
    rwe = lax.dot_general(rbft, We[...], dn_t, preferred_element_type=f32)
    mij = h * g * rwe
    enew_ref[...] = ef + mij

    F = jnp.dot(mij.astype(bf), Wfix[...], preferred_element_type=f32)
    p = _silu(Z[:, 2 * H:3 * H] + F[:, 0 * H:1 * H])
    p = _silu(jnp.dot(p.astype(bf), nW2[...], preferred_element_type=f32) + nb2[...])
    q = _silu(Z[:, 3 * H:4 * H] + F[:, 1 * H:2 * H])
    q = _sigmoid(jnp.dot(q.astype(bf), nG2[...], preferred_element_type=f32) + ng2[...])
    rwv = lax.dot_general(rbft, Wv[...], dn_t, preferred_element_type=f32)
    mess_ref[...] = p * q * rwv


def _mlp_body_alias(vi_ref, vj_ref, ef_ref, rbft_ref,
                    W1vi, W1vj, W1ef, b1, Wfix,
                    eW2, eb2, eG2, eg2, nW2, nb2, nG2, ng2,
                    We, Wv, eprev_ref, enew_ref, mess_ref):
    del eprev_ref
    _mlp_body(vi_ref, vj_ref, ef_ref, rbft_ref,
              W1vi, W1vj, W1ef, b1, Wfix,
              eW2, eb2, eG2, eg2, nW2, nb2, nG2, ng2,
              We, Wv, enew_ref, mess_ref)


def _fixed(i):
    return (0, 0)


def _tc_mlp_chunk(c, vivj, edge_feat, rbft, weights, e_prev):
    off = c * CBLK
    in_specs = [
        pl.BlockSpec((BE, D), lambda i: (i, 0)),
        pl.BlockSpec((BE, D), lambda i: (CBLK + i, 0)),
        pl.BlockSpec((BE, D), lambda i, off=off: (off + i, 0)),
        pl.BlockSpec((R, BE), lambda i, off=off: (0, off + i)),
        *[pl.BlockSpec(w.shape, _fixed) for w in weights],
    ]
    out_specs = [
        pl.BlockSpec((BE, D), lambda i, off=off: (off + i, 0)),
        pl.BlockSpec((BE, D), lambda i: (i, 0)),
    ]
    out_shape = [
        jax.ShapeDtypeStruct((E, D), jnp.float32),
        jax.ShapeDtypeStruct((EC, D), jnp.float32),
    ]
    args = [vivj, vivj, edge_feat, rbft, *weights]
    if e_prev is None:
        body = _mlp_body
        aliases = {}
    else:
        body = _mlp_body_alias
        in_specs.append(pl.BlockSpec(memory_space=pl.ANY))
        args.append(e_prev)
        aliases = {len(args) - 1: 0}
    return pl.pallas_call(
        body,
        grid=(CBLK,),
        in_specs=in_specs,
        out_specs=out_specs,
        out_shape=out_shape,
        input_output_aliases=aliases,
        compiler_params=pltpu.CompilerParams(
            dimension_semantics=("arbitrary",),
        ),
    )(*args)


SCH = 128
NSTEP = EC // SCH
SBASE = NSTEP // NW
SEXTRA = NSTEP - SBASE * NW
NPS = 632
N_PAD = NPS * NS


@functools.partial(
    pl.kernel,
    out_type=jax.ShapeDtypeStruct((2 * N_PAD, D), jnp.float32),
    mesh=_SC_MESH,
    scratch_types=[
        pltpu.VMEM((2, SCH), jnp.int32),
        pltpu.VMEM((2, SCH, D), jnp.float32),
        pltpu.VMEM_SHARED((N_PAD, D), jnp.float32),
        pltpu.SemaphoreType.DMA,
        pltpu.SemaphoreType.DMA,
    ],
)
def _sc_scatter(mess_hbm, dst_hbm, init_hbm, out_hbm, idx_v, rows_v,
                acc_sh, lsem, ssem):
    cid = lax.axis_index("c")
    sid = lax.axis_index("s")
    pltpu.sync_copy(init_hbm.at[pl.ds(cid * N_PAD + sid * NPS, NPS)],
                    acc_sh.at[pl.ds(sid * NPS, NPS)])
    plsc.subcore_barrier()

    wid = sid * NC + cid
    nsteps = jnp.where(wid < SEXTRA, SBASE + 1, SBASE)

    def step(k, carry):
        b = lax.rem(k, 2)
        pb = 1 - b
        off = (wid + k * NW) * SCH

        @pl.when(k >= 2)
        def _drain_scatter():
            pltpu.make_async_copy(
                rows_v.at[b], acc_sh.at[idx_v.at[b]], ssem).wait()

        @pl.when(k >= 1)
        def _retire_prev():
            pltpu.make_async_copy(
                mess_hbm.at[pl.ds(0, SCH)], rows_v.at[pb], lsem).wait()
            pltpu.async_copy(rows_v.at[pb], acc_sh.at[idx_v.at[pb]], ssem,
                             add=True)

        pltpu.sync_copy(dst_hbm.at[pl.ds(off, SCH)], idx_v.at[b])
        pltpu.async_copy(mess_hbm.at[pl.ds(off, SCH)], rows_v.at[b], lsem)
        return carry

    lax.fori_loop(0, nsteps, step, 0)
    lb = lax.rem(nsteps - 1, 2)
    pltpu.make_async_copy(mess_hbm.at[pl.ds(0, SCH)], rows_v.at[lb], lsem).wait()
    pltpu.async_copy(rows_v.at[lb], acc_sh.at[idx_v.at[lb]], ssem, add=True)
    pltpu.make_async_copy(rows_v.at[0], acc_sh.at[idx_v.at[0]], ssem).wait()
    pltpu.make_async_copy(rows_v.at[0], acc_sh.at[idx_v.at[0]], ssem).wait()

    plsc.subcore_barrier()
    pltpu.sync_copy(acc_sh.at[pl.ds(sid * NPS, NPS)],
                    out_hbm.at[pl.ds(cid * N_PAD + sid * NPS, NPS)])


def kernel(node_feat, edge_feat, rbf, edge_index,
           eW1, eb1, eW2, eb2, eG1, eg1, eG2, eg2,
           nW1, nb1, nW2, nb2, nG1, ng1, nG2, ng2,
           We, Wv):
    src = edge_index[0].astype(jnp.int32)
    dst = edge_index[1].astype(jnp.int32)
    bf = jnp.bfloat16

    w1 = jnp.concatenate([eW1, eG1, nW1, nG1], axis=1).astype(bf)
    b1 = jnp.concatenate([eb1, eg1, nb1, ng1]).reshape(1, 4 * H)
    wfix = jnp.concatenate([nW1[2 * D:], nG1[2 * D:]], axis=1).astype(bf)
    weights = (w1[:D], w1[D:2 * D], w1[2 * D:], b1, wfix,
               eW2.astype(bf), eb2.reshape(1, H), eG2.astype(bf), eg2.reshape(1, H),
               nW2.astype(bf), nb2.reshape(1, H), nG2.astype(bf), ng2.reshape(1, H),
               We, Wv)
    rbft = rbf.T

    vivjs = [
        _sc_gather(jnp.concatenate([src[c * EC:(c + 1) * EC],
                                    dst[c * EC:(c + 1) * EC]]), node_feat)
        for c in range(CH)
    ]

    part = jnp.zeros((2 * N_PAD, D), jnp.float32).at[:N].set(node_feat)
    e_new = None
    for c in range(CH):
        e_new, mess = _tc_mlp_chunk(c, vivjs[c], edge_feat, rbft, weights, e_new)
        part = _sc_scatter(mess, dst[c * EC:(c + 1) * EC], part)

    v_new = part[:N] + part[N_PAD:N_PAD + N]
    return (e_new, v_new)
